# Initial kernel scaffold; baseline (speedup 1.0000x reference)
import functools

import jax
import jax.numpy as jnp
from jax import lax
from jax.experimental import pallas as pl
from jax.experimental.pallas import tpu as pltpu
from jax.experimental.pallas import tpu_sc as plsc

N_NODES = 10000
N_EDGES = 320000
FEAT = 128
AUG = 144

NC = 2
NS = 16
EDGES_PER_TILE = N_EDGES // (NC * NS)
CHUNK = 80
N_CHUNKS = EDGES_PER_TILE // CHUNK
ROWS_PER_TILE = N_NODES // NS
WCHUNK = 125
N_WCHUNKS = ROWS_PER_TILE // WCHUNK

_ZCOLS = AUG // 16


def _sc_body(src_hbm, dst_hbm, feat_hbm, out_hbm,
             src_v, dst_v, rows_v, wout_v, acc_sh, sem):
    c = lax.axis_index("c")
    s = lax.axis_index("s")

    zvec = jnp.zeros((16,), jnp.float32)

    def zrow(i, carry):
        for j in range(_ZCOLS):
            wout_v[i, pl.ds(j * 16, 16)] = zvec
        return carry

    lax.fori_loop(0, WCHUNK, zrow, 0)
    for wch in range(N_WCHUNKS):
        r0 = s * ROWS_PER_TILE + wch * WCHUNK
        pltpu.sync_copy(wout_v, acc_sh.at[pl.ds(r0, WCHUNK)])
    plsc.subcore_barrier()

    base = (c * NS + s) * EDGES_PER_TILE

    def body(it, carry):
        off = pl.multiple_of(base + it * CHUNK, 8)
        pltpu.sync_copy(src_hbm.at[pl.ds(off, CHUNK)], src_v)
        pltpu.sync_copy(dst_hbm.at[pl.ds(off, CHUNK)], dst_v)
        pltpu.async_copy(feat_hbm.at[src_v], rows_v, sem).wait()
        pltpu.sync_copy(rows_v, acc_sh.at[dst_v], add=True)
        return carry

    lax.fori_loop(0, N_CHUNKS, body, 0)
    plsc.subcore_barrier()

    for wch in range(N_WCHUNKS):
        r0 = s * ROWS_PER_TILE + wch * WCHUNK
        pltpu.sync_copy(acc_sh.at[pl.ds(r0, WCHUNK)], wout_v)
        pltpu.sync_copy(wout_v, out_hbm.at[c, pl.ds(r0, WCHUNK)])


def _make_sc_scatter():
    mesh = plsc.VectorSubcoreMesh(core_axis_name="c", subcore_axis_name="s")
    return pl.kernel(
        _sc_body,
        mesh=mesh,
        out_type=jax.ShapeDtypeStruct((NC, N_NODES, AUG), jnp.float32),
        scratch_types=[
            pltpu.VMEM((CHUNK,), jnp.int32),
            pltpu.VMEM((CHUNK,), jnp.int32),
            pltpu.VMEM((CHUNK, AUG), jnp.float32),
            pltpu.VMEM((WCHUNK, AUG), jnp.float32),
            pltpu.VMEM_SHARED((N_NODES, AUG), jnp.float32),
            pltpu.SemaphoreType.DMA,
        ],
    )


_BLK = 1000


def _tc_body(self_ref, p0_ref, p1_ref, w_ref, out_ref, *, out_aug):
    acc = p0_ref[...] + p1_ref[...]
    deg = jnp.maximum(acc[:, FEAT:FEAT + 1], 1.0)
    aggn = acc[:, :FEAT] / deg
    h = (jnp.dot(self_ref[:, :FEAT], w_ref[:FEAT, :],
                 preferred_element_type=jnp.float32)
         + jnp.dot(aggn, w_ref[FEAT:, :], preferred_element_type=jnp.float32))
    if out_aug:
        tail = jnp.concatenate(
            [jnp.ones((_BLK, 1), jnp.float32),
             jnp.zeros((_BLK, AUG - FEAT - 1), jnp.float32)], axis=1)
        out_ref[...] = jnp.concatenate([h, tail], axis=1)
    else:
        out_ref[...] = h


def _make_tc_combine(out_aug):
    out_w = AUG if out_aug else FEAT
    return pl.pallas_call(
        functools.partial(_tc_body, out_aug=out_aug),
        grid=(N_NODES // _BLK,),
        in_specs=[
            pl.BlockSpec((_BLK, AUG), lambda i: (i, 0)),
            pl.BlockSpec((_BLK, AUG), lambda i: (i, 0)),
            pl.BlockSpec((_BLK, AUG), lambda i: (i, 0)),
            pl.BlockSpec((2 * FEAT, FEAT), lambda i: (0, 0)),
        ],
        out_specs=pl.BlockSpec((_BLK, out_w), lambda i: (i, 0)),
        out_shape=jax.ShapeDtypeStruct((N_NODES, out_w), jnp.float32),
    )


def kernel(x, edge_index, w1, w2):
    src = edge_index[0]
    dst = edge_index[1]
    x_aug = jnp.concatenate(
        [x, jnp.ones((N_NODES, 1), jnp.float32),
         jnp.zeros((N_NODES, AUG - FEAT - 1), jnp.float32)], axis=1)

    sc_scatter = _make_sc_scatter()
    tc_aug = _make_tc_combine(out_aug=True)
    tc_out = _make_tc_combine(out_aug=False)

    p1 = sc_scatter(src, dst, x_aug)
    h1_aug = tc_aug(x_aug, p1[0], p1[1], w1)
    p2 = sc_scatter(src, dst, h1_aug)
    return tc_out(h1_aug, p2[0], p2[1], w2)


# speedup vs baseline: 4.3863x; 4.3863x over previous
import functools

import jax
import jax.numpy as jnp
from jax import lax
from jax.experimental import pallas as pl
from jax.experimental.pallas import tpu as pltpu
from jax.experimental.pallas import tpu_sc as plsc

N_NODES = 10000
N_EDGES = 320000
FEAT = 128
AUG = 144

NC = 2
NS = 16
EDGES_PER_TILE = N_EDGES // (NC * NS)
CHUNK = 80
N_CHUNKS = EDGES_PER_TILE // CHUNK
N_PAD = 10240
ROWS_PER_TILE = N_PAD // NS
WCHUNK = 128
N_WCHUNKS = ROWS_PER_TILE // WCHUNK

_ZCOLS = AUG // 16


def _sc_body(src_hbm, dst_hbm, feat_hbm, out_hbm,
             src_v, dst_v, rows_v, wout_v, acc_sh, sem):
    c = lax.axis_index("c")
    s = lax.axis_index("s")

    zvec = jnp.zeros((16,), jnp.float32)

    def zrow(i, carry):
        for j in range(_ZCOLS):
            wout_v[i, pl.ds(j * 16, 16)] = zvec
        return carry

    lax.fori_loop(0, WCHUNK, zrow, 0)
    for wch in range(N_WCHUNKS):
        r0 = s * ROWS_PER_TILE + wch * WCHUNK
        pltpu.sync_copy(wout_v, acc_sh.at[pl.ds(r0, WCHUNK)])
    plsc.subcore_barrier()

    base = (c * NS + s) * EDGES_PER_TILE

    def body(it, carry):
        off = pl.multiple_of(base + it * CHUNK, 8)
        pltpu.sync_copy(src_hbm.at[pl.ds(off, CHUNK)], src_v)
        pltpu.sync_copy(dst_hbm.at[pl.ds(off, CHUNK)], dst_v)
        pltpu.async_copy(feat_hbm.at[src_v], rows_v, sem).wait()
        pltpu.sync_copy(rows_v, acc_sh.at[dst_v], add=True)
        return carry

    lax.fori_loop(0, N_CHUNKS, body, 0)
    plsc.subcore_barrier()

    for wch in range(N_WCHUNKS):
        r0 = s * ROWS_PER_TILE + wch * WCHUNK
        pltpu.sync_copy(acc_sh.at[pl.ds(r0, WCHUNK)], wout_v)
        pltpu.sync_copy(wout_v, out_hbm.at[c, pl.ds(r0, WCHUNK)])


def _make_sc_scatter():
    mesh = plsc.VectorSubcoreMesh(core_axis_name="c", subcore_axis_name="s")
    return pl.kernel(
        _sc_body,
        mesh=mesh,
        out_type=jax.ShapeDtypeStruct((NC, N_PAD, AUG), jnp.float32),
        scratch_types=[
            pltpu.VMEM((CHUNK,), jnp.int32),
            pltpu.VMEM((CHUNK,), jnp.int32),
            pltpu.VMEM((CHUNK, AUG), jnp.float32),
            pltpu.VMEM((WCHUNK, AUG), jnp.float32),
            pltpu.VMEM_SHARED((N_PAD, AUG), jnp.float32),
            pltpu.SemaphoreType.DMA,
        ],
        compiler_params=pltpu.CompilerParams(use_tc_tiling_on_sc=False),
    )


_BLK = 1000


def _tc_body(self_ref, p0_ref, p1_ref, w_ref, out_ref, *, out_aug):
    acc = p0_ref[...] + p1_ref[...]
    deg = jnp.maximum(acc[:, FEAT:FEAT + 1], 1.0)
    aggn = acc[:, :FEAT] / deg
    h = (jnp.dot(self_ref[:, :FEAT], w_ref[:FEAT, :],
                 preferred_element_type=jnp.float32)
         + jnp.dot(aggn, w_ref[FEAT:, :], preferred_element_type=jnp.float32))
    if out_aug:
        tail = jnp.concatenate(
            [jnp.ones((_BLK, 1), jnp.float32),
             jnp.zeros((_BLK, AUG - FEAT - 1), jnp.float32)], axis=1)
        out_ref[...] = jnp.concatenate([h, tail], axis=1)
    else:
        out_ref[...] = h


def _make_tc_combine(out_aug):
    out_w = AUG if out_aug else FEAT
    return pl.pallas_call(
        functools.partial(_tc_body, out_aug=out_aug),
        grid=(N_NODES // _BLK,),
        in_specs=[
            pl.BlockSpec((_BLK, AUG), lambda i: (i, 0)),
            pl.BlockSpec((_BLK, AUG), lambda i: (i, 0)),
            pl.BlockSpec((_BLK, AUG), lambda i: (i, 0)),
            pl.BlockSpec((2 * FEAT, FEAT), lambda i: (0, 0)),
        ],
        out_specs=pl.BlockSpec((_BLK, out_w), lambda i: (i, 0)),
        out_shape=jax.ShapeDtypeStruct((N_NODES, out_w), jnp.float32),
    )


def kernel(x, edge_index, w1, w2):
    src = edge_index[0]
    dst = edge_index[1]
    x_aug = jnp.concatenate(
        [x, jnp.ones((N_NODES, 1), jnp.float32),
         jnp.zeros((N_NODES, AUG - FEAT - 1), jnp.float32)], axis=1)

    sc_scatter = _make_sc_scatter()
    tc_aug = _make_tc_combine(out_aug=True)
    tc_out = _make_tc_combine(out_aug=False)

    p1 = sc_scatter(src, dst, x_aug)
    h1_aug = tc_aug(x_aug, p1[0], p1[1], w1)
    p2 = sc_scatter(src, dst, h1_aug)
    return tc_out(h1_aug, p2[0], p2[1], w2)


# speedup vs baseline: 8.7591x; 1.9969x over previous
import functools

import jax
import jax.numpy as jnp
from jax import lax
from jax.experimental import pallas as pl
from jax.experimental.pallas import tpu as pltpu
from jax.experimental.pallas import tpu_sc as plsc

N_NODES = 10000
N_EDGES = 320000
FEAT = 128
AUG = 144

NC = 2
NS = 16
EDGES_PER_TILE = N_EDGES // (NC * NS)
CHUNK = 80
N_CHUNKS = EDGES_PER_TILE // CHUNK
N_PAD = 10112
ROWS_PER_TILE = N_PAD // NS
WCHUNK = 80
WTAIL = ROWS_PER_TILE - 7 * WCHUNK

_ZCOLS = AUG // 16


def _sc_body(edges_hbm, feat_hbm, out_hbm,
             i0, i1, i2, i3, ra, rb, wout_v, acc_sh,
             is0, is1, is2, is3, ga, gb, sa, sb, wsem):
    c = lax.axis_index("c")
    s = lax.axis_index("s")
    islot = [i0, i1, i2, i3]
    isem = [is0, is1, is2, is3]
    rows = [ra, rb]
    gsem = [ga, gb]
    ssem = [sa, sb]
    cb = (c * NS + s) * N_CHUNKS

    def idx_load(it):
        return pltpu.async_copy(edges_hbm.at[cb + it], islot[it % 4],
                                isem[it % 4])

    def gather(it):
        return pltpu.async_copy(feat_hbm.at[islot[it % 4].at[0]],
                                rows[it % 2], gsem[it % 2])

    zvec = jnp.zeros((16,), jnp.float32)

    def zrow(i, carry):
        for j in range(_ZCOLS):
            wout_v[i, pl.ds(j * 16, 16)] = zvec
        return carry

    lax.fori_loop(0, WCHUNK, zrow, 0)
    rbase = s * ROWS_PER_TILE
    zcps = [pltpu.async_copy(wout_v,
                             acc_sh.at[pl.ds(rbase + k * WCHUNK, WCHUNK)],
                             wsem)
            for k in range(7)]
    zcps.append(pltpu.async_copy(wout_v.at[pl.ds(0, WTAIL)],
                                 acc_sh.at[pl.ds(rbase + 7 * WCHUNK, WTAIL)],
                                 wsem))
    for cp in zcps:
        cp.wait()
    plsc.subcore_barrier()

    idx_load(0)
    idx_load(1)
    pltpu.make_async_copy(edges_hbm.at[cb + 0], islot[0], isem[0]).wait()
    gather(0)
    for it in range(N_CHUNKS):
        if it + 1 < N_CHUNKS:
            pltpu.make_async_copy(edges_hbm.at[cb + it + 1],
                                  islot[(it + 1) % 4],
                                  isem[(it + 1) % 4]).wait()
            if it >= 1:
                pltpu.make_async_copy(rows[(it + 1) % 2],
                                      acc_sh.at[islot[(it - 1) % 4].at[1]],
                                      ssem[(it + 1) % 2]).wait()
            gather(it + 1)
        pltpu.make_async_copy(feat_hbm.at[islot[it % 4].at[0]],
                              rows[it % 2], gsem[it % 2]).wait()
        pltpu.async_copy(rows[it % 2], acc_sh.at[islot[it % 4].at[1]],
                         ssem[it % 2], add=True)
        if it + 2 < N_CHUNKS:
            idx_load(it + 2)
    for it in (N_CHUNKS - 2, N_CHUNKS - 1):
        pltpu.make_async_copy(rows[it % 2], acc_sh.at[islot[it % 4].at[1]],
                              ssem[it % 2]).wait()
    plsc.subcore_barrier()

    for k in range(7):
        r0_ = rbase + k * WCHUNK
        pltpu.sync_copy(acc_sh.at[pl.ds(r0_, WCHUNK)], wout_v)
        pltpu.sync_copy(wout_v, out_hbm.at[c, pl.ds(r0_, WCHUNK)])
    r0_ = rbase + 7 * WCHUNK
    pltpu.sync_copy(acc_sh.at[pl.ds(r0_, WTAIL)],
                    wout_v.at[pl.ds(0, WTAIL)])
    pltpu.sync_copy(wout_v.at[pl.ds(0, WTAIL)],
                    out_hbm.at[c, pl.ds(r0_, WTAIL)])


def _make_sc_scatter():
    mesh = plsc.VectorSubcoreMesh(core_axis_name="c", subcore_axis_name="s")
    return pl.kernel(
        _sc_body,
        mesh=mesh,
        out_type=jax.ShapeDtypeStruct((NC, N_PAD, AUG), jnp.float32),
        scratch_types=[
            pltpu.VMEM((2, CHUNK), jnp.int32),
            pltpu.VMEM((2, CHUNK), jnp.int32),
            pltpu.VMEM((2, CHUNK), jnp.int32),
            pltpu.VMEM((2, CHUNK), jnp.int32),
            pltpu.VMEM((CHUNK, AUG), jnp.float32),
            pltpu.VMEM((CHUNK, AUG), jnp.float32),
            pltpu.VMEM((WCHUNK, AUG), jnp.float32),
            pltpu.VMEM_SHARED((N_PAD, AUG), jnp.float32),
        ] + [pltpu.SemaphoreType.DMA for _ in range(9)],
        compiler_params=pltpu.CompilerParams(use_tc_tiling_on_sc=False),
    )


_BLK = 1000


def _tc_body(self_ref, p0_ref, p1_ref, w_ref, out_ref, *, out_aug):
    acc = p0_ref[0] + p1_ref[0]
    deg = jnp.maximum(acc[:, FEAT:FEAT + 1], 1.0)
    aggn = acc[:, :FEAT] / deg
    h = (jnp.dot(self_ref[:, :FEAT], w_ref[:FEAT, :],
                 preferred_element_type=jnp.float32)
         + jnp.dot(aggn, w_ref[FEAT:, :], preferred_element_type=jnp.float32))
    if out_aug:
        tail = jnp.concatenate(
            [jnp.ones((_BLK, 1), jnp.float32),
             jnp.zeros((_BLK, AUG - FEAT - 1), jnp.float32)], axis=1)
        out_ref[...] = jnp.concatenate([h, tail], axis=1)
    else:
        out_ref[...] = h


def _make_tc_combine(out_aug):
    out_w = AUG if out_aug else FEAT
    return pl.pallas_call(
        functools.partial(_tc_body, out_aug=out_aug),
        grid=(N_NODES // _BLK,),
        in_specs=[
            pl.BlockSpec((_BLK, AUG), lambda i: (i, 0)),
            pl.BlockSpec((1, _BLK, AUG), lambda i: (0, i, 0)),
            pl.BlockSpec((1, _BLK, AUG), lambda i: (1, i, 0)),
            pl.BlockSpec((2 * FEAT, FEAT), lambda i: (0, 0)),
        ],
        out_specs=pl.BlockSpec((_BLK, out_w), lambda i: (i, 0)),
        out_shape=jax.ShapeDtypeStruct((N_NODES, out_w), jnp.float32),
    )


def kernel(x, edge_index, w1, w2):
    edges_r = jnp.transpose(
        edge_index.reshape(2, NC * NS * N_CHUNKS, CHUNK), (1, 0, 2))
    x_aug = jnp.concatenate(
        [x, jnp.ones((N_NODES, 1), jnp.float32),
         jnp.zeros((N_NODES, AUG - FEAT - 1), jnp.float32)], axis=1)

    sc_scatter = _make_sc_scatter()
    tc_aug = _make_tc_combine(out_aug=True)
    tc_out = _make_tc_combine(out_aug=False)

    p1 = sc_scatter(edges_r, x_aug)
    h1_aug = tc_aug(x_aug, p1, p1, w1)
    p2 = sc_scatter(edges_r, h1_aug)
    return tc_out(h1_aug, p2, p2, w2)


# speedup vs baseline: 12.1173x; 1.3834x over previous
import jax
import jax.numpy as jnp
from jax import lax
from jax.experimental import pallas as pl
from jax.experimental.pallas import tpu as pltpu
from jax.experimental.pallas import tpu_sc as plsc

N_NODES = 10000
N_EDGES = 320000
FEAT = 128

NC = 2
NS = 16
NT = NC * NS
CHUNK = 128
N_CHUNK_ROWS = N_EDGES // CHUNK
CHUNKS_PER_TILE = N_CHUNK_ROWS // NT
N_LEFTOVER = N_CHUNK_ROWS - NT * CHUNKS_PER_TILE
N_PAD = 10112
ROWS_PER_TILE = N_PAD // NS
WCHUNK = 80
WTAIL = ROWS_PER_TILE - 7 * WCHUNK


def _sc_body(srcr_hbm, dstr_hbm, feat_hbm, out_hbm, hist_hbm,
             si0, si1, si2, si3, di0, di1, di2, di3, ra, rb, hist_v, acc_sh,
             is0, is1, is2, is3, ga, gb, sa, sb, wsem):
    c = lax.axis_index("c")
    s = lax.axis_index("s")
    t = c * NS + s
    sslot = [si0, si1, si2, si3]
    dslot = [di0, di1, di2, di3]
    isem = [is0, is1, is2, is3]
    rows = [ra, rb]
    gsem = [ga, gb]
    ssem = [sa, sb]
    cb = t * CHUNKS_PER_TILE

    def idx_load(row, k):
        pltpu.async_copy(srcr_hbm.at[row], sslot[k], isem[k])
        pltpu.async_copy(dstr_hbm.at[row], dslot[k], isem[k])

    def idx_wait(row, k):
        pltpu.make_async_copy(srcr_hbm.at[row], sslot[k], isem[k]).wait()
        pltpu.make_async_copy(dstr_hbm.at[row], dslot[k], isem[k]).wait()

    def gather(k, b):
        pltpu.async_copy(feat_hbm.at[sslot[k]], rows[b], gsem[b])

    def gather_wait(k, b):
        pltpu.make_async_copy(feat_hbm.at[sslot[k]], rows[b],
                              gsem[b]).wait()

    def scatter(k, b):
        pltpu.async_copy(rows[b], acc_sh.at[dslot[k]], ssem[b], add=True)

    def scatter_wait(k, b):
        pltpu.make_async_copy(rows[b], acc_sh.at[dslot[k]],
                              ssem[b]).wait()

    ones16 = jnp.ones((16,), jnp.float32)

    def hist_update(k):
        for j in range(CHUNK // 16):
            d = dslot[k][pl.ds(j * 16, 16)]
            plsc.addupdate_scatter(hist_v, [d], ones16)

    zvec = jnp.zeros((16,), jnp.float32)

    def zhist(i, carry):
        hist_v[pl.ds(i * 16, 16)] = zvec
        return carry

    lax.fori_loop(0, N_PAD // 16, zhist, 0)

    def zrow(i, carry):
        for j in range(FEAT // 16):
            ra[i, pl.ds(j * 16, 16)] = zvec
        return carry

    lax.fori_loop(0, CHUNK, zrow, 0)
    rbase = s * ROWS_PER_TILE
    zcps = [pltpu.async_copy(ra.at[pl.ds(0, WCHUNK)],
                             acc_sh.at[pl.ds(rbase + k * WCHUNK, WCHUNK)],
                             wsem)
            for k in range(7)]
    zcps.append(pltpu.async_copy(ra.at[pl.ds(0, WTAIL)],
                                 acc_sh.at[pl.ds(rbase + 7 * WCHUNK, WTAIL)],
                                 wsem))
    for cp in zcps:
        cp.wait()
    plsc.subcore_barrier()

    idx_load(cb + 0, 0)
    idx_load(cb + 1, 1)
    idx_wait(cb + 0, 0)
    gather(0, 0)
    for it in range(CHUNKS_PER_TILE):
        k, b = it % 4, it % 2
        if it + 1 < CHUNKS_PER_TILE:
            idx_wait(cb + it + 1, (it + 1) % 4)
            if it >= 1:
                scatter_wait((it - 1) % 4, (it - 1) % 2)
            gather((it + 1) % 4, (it + 1) % 2)
        gather_wait(k, b)
        scatter(k, b)
        hist_update(k)
        if it + 2 < CHUNKS_PER_TILE:
            idx_load(cb + it + 2, (it + 2) % 4)
    for it in (CHUNKS_PER_TILE - 2, CHUNKS_PER_TILE - 1):
        scatter_wait(it % 4, it % 2)

    @pl.when(t < N_LEFTOVER)
    def _():
        row = NT * CHUNKS_PER_TILE + t
        idx_load(row, 0)
        idx_wait(row, 0)
        gather(0, 0)
        gather_wait(0, 0)
        scatter(0, 0)
        hist_update(0)
        scatter_wait(0, 0)

    plsc.subcore_barrier()

    for k in range(7):
        r0_ = rbase + k * WCHUNK
        pltpu.sync_copy(acc_sh.at[pl.ds(r0_, WCHUNK)],
                        ra.at[pl.ds(0, WCHUNK)])
        pltpu.sync_copy(ra.at[pl.ds(0, WCHUNK)],
                        out_hbm.at[c, pl.ds(r0_, WCHUNK)])
    r0_ = rbase + 7 * WCHUNK
    pltpu.sync_copy(acc_sh.at[pl.ds(r0_, WTAIL)], ra.at[pl.ds(0, WTAIL)])
    pltpu.sync_copy(ra.at[pl.ds(0, WTAIL)], out_hbm.at[c, pl.ds(r0_, WTAIL)])
    pltpu.sync_copy(hist_v, hist_hbm.at[c, s])


def _make_sc_scatter():
    mesh = plsc.VectorSubcoreMesh(core_axis_name="c", subcore_axis_name="s")
    return pl.kernel(
        _sc_body,
        mesh=mesh,
        out_type=(
            jax.ShapeDtypeStruct((NC, N_PAD, FEAT), jnp.float32),
            jax.ShapeDtypeStruct((NC, NS, N_PAD), jnp.float32),
        ),
        scratch_types=[
            pltpu.VMEM((CHUNK,), jnp.int32),
            pltpu.VMEM((CHUNK,), jnp.int32),
            pltpu.VMEM((CHUNK,), jnp.int32),
            pltpu.VMEM((CHUNK,), jnp.int32),
            pltpu.VMEM((CHUNK,), jnp.int32),
            pltpu.VMEM((CHUNK,), jnp.int32),
            pltpu.VMEM((CHUNK,), jnp.int32),
            pltpu.VMEM((CHUNK,), jnp.int32),
            pltpu.VMEM((CHUNK, FEAT), jnp.float32),
            pltpu.VMEM((CHUNK, FEAT), jnp.float32),
            pltpu.VMEM((N_PAD,), jnp.float32),
            pltpu.VMEM_SHARED((N_PAD, FEAT), jnp.float32),
        ] + [pltpu.SemaphoreType.DMA for _ in range(9)],
        compiler_params=pltpu.CompilerParams(use_tc_tiling_on_sc=True,
                                             needs_layout_passes=False),
    )


_BLK = 1000


def _tc_body(self_ref, p0_ref, p1_ref, deg_ref, w_ref, out_ref):
    acc = p0_ref[0] + p1_ref[0]
    deg = jnp.maximum(deg_ref[...], 1.0)
    aggn = acc / deg
    out_ref[...] = (
        jnp.dot(self_ref[...], w_ref[:FEAT, :],
                preferred_element_type=jnp.float32)
        + jnp.dot(aggn, w_ref[FEAT:, :], preferred_element_type=jnp.float32))


def _make_tc_combine():
    return pl.pallas_call(
        _tc_body,
        grid=(N_NODES // _BLK,),
        in_specs=[
            pl.BlockSpec((_BLK, FEAT), lambda i: (i, 0)),
            pl.BlockSpec((1, _BLK, FEAT), lambda i: (0, i, 0)),
            pl.BlockSpec((1, _BLK, FEAT), lambda i: (1, i, 0)),
            pl.BlockSpec((_BLK, 1), lambda i: (i, 0)),
            pl.BlockSpec((2 * FEAT, FEAT), lambda i: (0, 0)),
        ],
        out_specs=pl.BlockSpec((_BLK, FEAT), lambda i: (i, 0)),
        out_shape=jax.ShapeDtypeStruct((N_NODES, FEAT), jnp.float32),
    )


def kernel(x, edge_index, w1, w2):
    src_r = edge_index[0].reshape(N_CHUNK_ROWS, CHUNK)
    dst_r = edge_index[1].reshape(N_CHUNK_ROWS, CHUNK)

    sc_scatter = _make_sc_scatter()
    tc_combine = _make_tc_combine()

    p1, hist1 = sc_scatter(src_r, dst_r, x)
    deg = hist1.sum(axis=(0, 1)).reshape(N_PAD, 1)
    h1 = tc_combine(x, p1, p1, deg, w1)
    p2, _ = sc_scatter(src_r, dst_r, h1)
    return tc_combine(h1, p2, p2, deg, w2)
